# baseline (device time: 23464 ns/iter reference)
import jax
import jax.numpy as jnp
from jax import lax
from jax.experimental import pallas as pl
from jax.experimental.pallas import tpu as pltpu

N_DEV = 32
N_TOK = 256
D_IN = 128
D_OUT = 256
N_EXP = 64
ROWS = N_TOK // N_DEV


def kernel(x, router_W, route_idx, expert_W):
    def body(x_ref, rw_ref, idx_ref, ew_ref, out_ref,
             partial_ref, acc_ref, send_sems, recv_sems):
        my = lax.axis_index("i")

        xv = x_ref[...]
        scores = jnp.dot(xv, rw_ref[...], preferred_element_type=jnp.float32)
        m = jnp.max(scores, axis=-1, keepdims=True)
        p = jnp.exp(scores - m)
        p = p / jnp.sum(p, axis=-1, keepdims=True)
        e0 = idx_ref[:, 0:1]
        e1 = idx_ref[:, 1:2]
        lane = lax.broadcasted_iota(jnp.int32, (N_TOK, N_EXP), 1)
        g0 = jnp.sum(jnp.where(lane == e0, p, 0.0), axis=-1, keepdims=True)
        g1 = jnp.sum(jnp.where(lane == e1, p, 0.0), axis=-1, keepdims=True)
        gs = g0 + g1

        acc = jnp.zeros((N_TOK, D_OUT), jnp.float32)
        for k in range(2):
            gid = my * 2 + k
            w = (jnp.where(e0 == gid, g0, 0.0)
                 + jnp.where(e1 == gid, g1, 0.0)) / gs
            xe = (xv * w).astype(jnp.bfloat16)
            acc = acc + jnp.dot(xe, ew_ref[k].astype(jnp.bfloat16),
                                preferred_element_type=jnp.float32)
        partial_ref[...] = acc.reshape(N_DEV, ROWS, D_OUT)

        rdmas = []
        for o in range(1, N_DEV):
            dst = lax.rem(my + o, N_DEV)
            rdma = pltpu.make_async_remote_copy(
                src_ref=partial_ref.at[dst],
                dst_ref=acc_ref.at[o],
                send_sem=send_sems.at[o],
                recv_sem=recv_sems.at[o],
                device_id=(dst,),
                device_id_type=pl.DeviceIdType.MESH,
            )
            rdma.start()
            rdmas.append(rdma)

        acc_ref[0] = partial_ref[my]

        for rdma in rdmas:
            rdma.wait_recv()
        out_ref[...] = jnp.sum(acc_ref[...], axis=0)
        for rdma in rdmas:
            rdma.wait_send()

    return pl.pallas_call(
        body,
        out_shape=jax.ShapeDtypeStruct((ROWS, D_OUT), jnp.float32),
        in_specs=[pl.BlockSpec(memory_space=pltpu.VMEM)] * 4,
        out_specs=pl.BlockSpec(memory_space=pltpu.VMEM),
        scratch_shapes=[
            pltpu.VMEM((N_DEV, ROWS, D_OUT), jnp.float32),
            pltpu.VMEM((N_DEV, ROWS, D_OUT), jnp.float32),
            pltpu.SemaphoreType.DMA((N_DEV,)),
            pltpu.SemaphoreType.DMA((N_DEV,)),
        ],
    )(x, router_W, route_idx, expert_W)


# device time: 15594 ns/iter; 1.5047x vs baseline; 1.5047x over previous
import jax
import jax.numpy as jnp
from jax import lax
from jax.experimental import pallas as pl
from jax.experimental.pallas import tpu as pltpu

N_DEV = 32
N_TOK = 256
D_IN = 128
D_OUT = 256
N_EXP = 64
ROWS = N_TOK // N_DEV


def kernel(x, router_W, route_idx, expert_W):
    def body(x_ref, rw_ref, idx_ref, ew_ref, out_ref,
             partial_ref, acc_ref, send_sems, recv_sems):
        my = lax.axis_index("i")

        xv = x_ref[...]
        scores = jnp.dot(xv, rw_ref[...], preferred_element_type=jnp.float32)
        m = jnp.max(scores, axis=-1, keepdims=True)
        p = jnp.exp(scores - m)
        p = p / jnp.sum(p, axis=-1, keepdims=True)
        e0 = idx_ref[:, 0:1]
        e1 = idx_ref[:, 1:2]
        lane = lax.broadcasted_iota(jnp.int32, (N_TOK, N_EXP), 1)
        g0 = jnp.sum(jnp.where(lane == e0, p, 0.0), axis=-1, keepdims=True)
        g1 = jnp.sum(jnp.where(lane == e1, p, 0.0), axis=-1, keepdims=True)
        gs = g0 + g1

        acc = jnp.zeros((N_TOK, D_OUT), jnp.float32)
        for k in range(2):
            gid = my * 2 + k
            w = (jnp.where(e0 == gid, g0, 0.0)
                 + jnp.where(e1 == gid, g1, 0.0)) / gs
            xe = (xv * w).astype(jnp.bfloat16)
            acc = acc + jnp.dot(xe, ew_ref[k].astype(jnp.bfloat16),
                                preferred_element_type=jnp.float32)
        partial_ref[...] = acc.astype(jnp.bfloat16).reshape(N_DEV, ROWS, D_OUT)

        barrier = pltpu.get_barrier_semaphore()
        for o in range(1, N_DEV):
            pl.semaphore_signal(
                barrier, inc=1,
                device_id=(lax.rem(my + o, N_DEV),),
                device_id_type=pl.DeviceIdType.MESH,
            )
        pl.semaphore_wait(barrier, N_DEV - 1)

        rdmas = []
        for o in range(1, N_DEV):
            dst = lax.rem(my + o, N_DEV)
            rdma = pltpu.make_async_remote_copy(
                src_ref=partial_ref.at[dst],
                dst_ref=acc_ref.at[o],
                send_sem=send_sems.at[o],
                recv_sem=recv_sems.at[o],
                device_id=(dst,),
                device_id_type=pl.DeviceIdType.MESH,
            )
            rdma.start()
            rdmas.append(rdma)

        acc_ref[0] = partial_ref[my]

        for rdma in rdmas:
            rdma.wait_recv()
        out_ref[...] = jnp.sum(acc_ref[...].astype(jnp.float32), axis=0)
        for rdma in rdmas:
            rdma.wait_send()

    return pl.pallas_call(
        body,
        out_shape=jax.ShapeDtypeStruct((ROWS, D_OUT), jnp.float32),
        in_specs=[pl.BlockSpec(memory_space=pltpu.VMEM)] * 4,
        out_specs=pl.BlockSpec(memory_space=pltpu.VMEM),
        scratch_shapes=[
            pltpu.VMEM((N_DEV, ROWS, D_OUT), jnp.bfloat16),
            pltpu.VMEM((N_DEV, ROWS, D_OUT), jnp.bfloat16),
            pltpu.SemaphoreType.DMA((N_DEV,)),
            pltpu.SemaphoreType.DMA((N_DEV,)),
        ],
        compiler_params=pltpu.CompilerParams(collective_id=0),
    )(x, router_W, route_idx, expert_W)


# device time: 14383 ns/iter; 1.6314x vs baseline; 1.0842x over previous
import jax
import jax.numpy as jnp
from jax import lax
from jax.experimental import pallas as pl
from jax.experimental.pallas import tpu as pltpu

N_DEV = 32
N_TOK = 256
D_IN = 128
D_OUT = 256
N_EXP = 64
ROWS = N_TOK // N_DEV


def kernel(x, router_W, route_idx, expert_W):
    def body(x_ref, rw_ref, idx_ref, ew_ref, out_ref,
             partial_ref, acc_ref, send_sems, recv_sems):
        my = lax.axis_index("i")

        barrier = pltpu.get_barrier_semaphore()
        for o in range(1, N_DEV):
            pl.semaphore_signal(
                barrier, inc=1,
                device_id=(lax.rem(my + o, N_DEV),),
                device_id_type=pl.DeviceIdType.MESH,
            )

        xv = x_ref[...]
        scores = jnp.dot(xv, rw_ref[...], preferred_element_type=jnp.float32)
        m = jnp.max(scores, axis=-1, keepdims=True)
        p = jnp.exp(scores - m)
        p = p / jnp.sum(p, axis=-1, keepdims=True)
        e0 = idx_ref[:, 0:1]
        e1 = idx_ref[:, 1:2]
        lane = lax.broadcasted_iota(jnp.int32, (N_TOK, N_EXP), 1)
        g0 = jnp.sum(jnp.where(lane == e0, p, 0.0), axis=-1, keepdims=True)
        g1 = jnp.sum(jnp.where(lane == e1, p, 0.0), axis=-1, keepdims=True)
        gs = g0 + g1

        acc = jnp.zeros((N_TOK, D_OUT), jnp.float32)
        for k in range(2):
            gid = my * 2 + k
            w = (jnp.where(e0 == gid, g0, 0.0)
                 + jnp.where(e1 == gid, g1, 0.0)) / gs
            xe = (xv * w).astype(jnp.bfloat16)
            acc = acc + jnp.dot(xe, ew_ref[k].astype(jnp.bfloat16),
                                preferred_element_type=jnp.float32)
        partial_ref[...] = acc.astype(jnp.bfloat16).reshape(N_DEV, ROWS, D_OUT)

        pl.semaphore_wait(barrier, N_DEV - 1)

        rdmas = []
        for o in range(1, N_DEV):
            dst = lax.rem(my + o, N_DEV)
            rdma = pltpu.make_async_remote_copy(
                src_ref=partial_ref.at[dst],
                dst_ref=acc_ref.at[o],
                send_sem=send_sems.at[o],
                recv_sem=recv_sems.at[o],
                device_id=(dst,),
                device_id_type=pl.DeviceIdType.MESH,
            )
            rdma.start()
            rdmas.append(rdma)

        acc_ref[0] = partial_ref[my]

        for rdma in rdmas:
            rdma.wait_recv()
        out_ref[...] = jnp.sum(acc_ref[...].astype(jnp.float32), axis=0)
        for rdma in rdmas:
            rdma.wait_send()

    return pl.pallas_call(
        body,
        out_shape=jax.ShapeDtypeStruct((ROWS, D_OUT), jnp.float32),
        in_specs=[pl.BlockSpec(memory_space=pltpu.VMEM)] * 4,
        out_specs=pl.BlockSpec(memory_space=pltpu.VMEM),
        scratch_shapes=[
            pltpu.VMEM((N_DEV, ROWS, D_OUT), jnp.bfloat16),
            pltpu.VMEM((N_DEV, ROWS, D_OUT), jnp.bfloat16),
            pltpu.SemaphoreType.DMA((N_DEV,)),
            pltpu.SemaphoreType.DMA((N_DEV,)),
        ],
        compiler_params=pltpu.CompilerParams(collective_id=0),
    )(x, router_W, route_idx, expert_W)
